# Initial kernel scaffold; baseline (speedup 1.0000x reference)
import jax
import jax.numpy as jnp
from jax.experimental import pallas as pl


def kernel(x, idxs, w, b):
    raise NotImplementedError("write your pallas kernel here")


# speedup vs baseline: 2.8403x; 2.8403x over previous
import functools

import jax
import jax.numpy as jnp
from jax import lax
from jax.experimental import pallas as pl
from jax.experimental.pallas import tpu as pltpu
from jax.experimental.pallas import tpu_sc as plsc

N_TOK = 8192
D_IN = 2048
D_OUT = 2048
N_EXP = 8

TM = 256
NT = N_TOK // TM
WMAX = NT + N_EXP - 1

_SC_NC = 2
_SC_NS = 16
_NW = _SC_NC * _SC_NS
_ROWS_PER_W = N_TOK // _NW
_CH = 32


@functools.lru_cache(maxsize=None)
def _make_row_gather(n_cols):
  mesh = plsc.VectorSubcoreMesh(
      core_axis_name="c", subcore_axis_name="s",
      num_cores=_SC_NC, num_subcores=_SC_NS)

  @functools.partial(
      pl.kernel,
      out_type=jax.ShapeDtypeStruct((N_TOK, n_cols), jnp.float32),
      mesh=mesh,
      scratch_types=[
          pltpu.VMEM((_CH,), jnp.int32),
          pltpu.VMEM((_CH, n_cols), jnp.float32),
          pltpu.SemaphoreType.DMA,
      ],
  )
  def gather_kernel(src_hbm, idx_hbm, out_hbm, idx_v, rows_v, sem):
    wid = lax.axis_index("s") * _SC_NC + lax.axis_index("c")
    base = wid * _ROWS_PER_W

    def chunk(k, carry):
      off = base + k * _CH
      pltpu.sync_copy(idx_hbm.at[pl.ds(off, _CH)], idx_v)
      pltpu.async_copy(src_hbm.at[idx_v], rows_v, sem).wait()
      pltpu.sync_copy(rows_v, out_hbm.at[pl.ds(off, _CH)])
      return carry

    lax.fori_loop(0, _ROWS_PER_W // _CH, chunk, 0)

  return gather_kernel


def _mm_body(m_ref, e_ref, lo_ref, hi_ref, xs_ref, w_ref, b_ref, out_ref):
  i = pl.program_id(0)
  lo = lo_ref[i]
  hi = hi_ref[i]

  @pl.when(hi > lo)
  def _():
    acc = jnp.dot(xs_ref[...], w_ref[0], preferred_element_type=jnp.float32)
    val = jnp.maximum(acc + b_ref[0, 0][None, :], 0.0)
    rows = lax.broadcasted_iota(jnp.int32, (TM, 1), 0)
    mask = (rows >= lo) & (rows < hi)
    out_ref[...] = jnp.where(mask, val, out_ref[...])


def _grouped_matmul(xs, w, b, m_of_w, e_of_w, lo_w, hi_w):
  grid_spec = pltpu.PrefetchScalarGridSpec(
      num_scalar_prefetch=4,
      grid=(WMAX,),
      in_specs=[
          pl.BlockSpec((TM, D_IN), lambda i, m, e, lo, hi: (m[i], 0)),
          pl.BlockSpec((1, D_IN, D_OUT), lambda i, m, e, lo, hi: (e[i], 0, 0)),
          pl.BlockSpec((1, 1, D_OUT), lambda i, m, e, lo, hi: (e[i], 0, 0)),
      ],
      out_specs=pl.BlockSpec((TM, D_OUT), lambda i, m, e, lo, hi: (m[i], 0)),
  )
  return pl.pallas_call(
      _mm_body,
      grid_spec=grid_spec,
      out_shape=jax.ShapeDtypeStruct((N_TOK, D_OUT), jnp.float32),
      compiler_params=pltpu.CompilerParams(
          dimension_semantics=("arbitrary",),
          vmem_limit_bytes=100 * 1024 * 1024,
      ),
  )(m_of_w, e_of_w, lo_w, hi_w, xs, w, b.reshape(N_EXP, 1, D_OUT))


def _schedule(e32):
  perm = jnp.argsort(e32, stable=True).astype(jnp.int32)
  pos = jnp.argsort(perm, stable=True).astype(jnp.int32)

  counts = jnp.bincount(e32, length=N_EXP).astype(jnp.int32)
  ends = jnp.cumsum(counts)
  starts = ends - counts
  nonempty = counts > 0
  t0 = jnp.where(nonempty, starts // TM, 0)
  t1 = jnp.where(nonempty, (ends - 1) // TM, -1)
  ntiles = jnp.where(nonempty, t1 - t0 + 1, 0)
  wstart = jnp.concatenate(
      [jnp.zeros((1,), jnp.int32), jnp.cumsum(ntiles).astype(jnp.int32)])
  n_work = wstart[-1]

  wids = jnp.arange(WMAX, dtype=jnp.int32)
  valid = wids < n_work
  e_of_w = jnp.minimum(
      jnp.sum((wids[:, None] >= wstart[None, 1:]).astype(jnp.int32), axis=1),
      N_EXP - 1)
  e_last = jnp.max(jnp.where(nonempty, jnp.arange(N_EXP, dtype=jnp.int32), 0))
  e_of_w = jnp.where(valid, e_of_w, e_last)
  m_of_w = jnp.where(valid, t0[e_of_w] + (wids - wstart[e_of_w]), NT - 1)
  lo_w = jnp.where(valid, jnp.clip(starts[e_of_w] - m_of_w * TM, 0, TM), 0)
  hi_w = jnp.where(valid, jnp.clip(ends[e_of_w] - m_of_w * TM, 0, TM), 0)
  return perm, pos, m_of_w, e_of_w, lo_w.astype(jnp.int32), hi_w.astype(jnp.int32)


def kernel(x, idxs, w, b):
  e32 = idxs.astype(jnp.int32)
  perm, pos, m_of_w, e_of_w, lo_w, hi_w = _schedule(e32)
  xs = _make_row_gather(D_IN)(x, perm)
  ys = _grouped_matmul(xs, w, b, m_of_w, e_of_w, lo_w, hi_w)
  return _make_row_gather(D_OUT)(ys, pos)
